# baseline (speedup 1.0000x reference)
import jax
import jax.numpy as jnp
from jax import lax
from jax.experimental import pallas as pl
from jax.experimental.pallas import tpu as pltpu

CIN = 5
KSIZE = 3
EPS = 1e-5
ENT_TILE = 4096


def _fused_kernel(band_ref, trip_ref,
                  t1_ref, t2_ref, his_ref,
                  w_fc1_ref, b_fc1_ref,
                  p0_ref, p0t_ref, g0_ref, b0_ref,
                  p1_ref, p1t_ref, g1_ref, b1_ref,
                  w_fc_ref, b_fc_ref, g2_ref, b2_ref,
                  emb_any, rel_any,
                  emb_ref, pmask_ref,
                  o_ref,
                  e1_scr, rel_scr, x_scr, wc_scr, dma_sem):
    j = pl.program_id(0)
    B, D = t1_ref.shape
    C = p1_ref.shape[1]

    @pl.when(j == 0)
    def conv_path():
        for c in range(C):
            for i in range(CIN):
                for k in range(KSIZE):
                    pltpu.make_async_copy(
                        band_ref.at[pl.ds(i * D + k, 1), pl.ds(c * D, 128)],
                        wc_scr.at[pl.ds(c, 1), pl.ds((3 * i + k) * 128, 128)],
                        dma_sem).start()
        def issue(i, carry):
            pltpu.make_async_copy(
                emb_any.at[pl.ds(trip_ref[i, 0], 1)],
                e1_scr.at[pl.ds(i, 1)], dma_sem).start()
            pltpu.make_async_copy(
                rel_any.at[pl.ds(trip_ref[i, 1], 1)],
                rel_scr.at[pl.ds(i, 1)], dma_sem).start()
            return carry
        lax.fori_loop(0, B, issue, 0)

        ones_row = jnp.ones((1, B), jnp.float32)
        his = jnp.dot(his_ref[...], w_fc1_ref[...],
                      preferred_element_type=jnp.float32) + b_fc1_ref[...]

        pltpu.make_async_copy(emb_any.at[pl.ds(0, B)], e1_scr, dma_sem).wait()
        pltpu.make_async_copy(rel_any.at[pl.ds(0, B)], rel_scr, dma_sem).wait()
        pltpu.make_async_copy(
            band_ref.at[pl.ds(0, 16), pl.ds(0, KSIZE * CIN * 128)], wc_scr,
            dma_sem).wait()
        x5 = jnp.concatenate(
            [jnp.tanh(e1_scr[...]), rel_scr[...], t1_ref[...], t2_ref[...],
             his], axis=1)

        def batch_norm(x, pool, poolt, gamma, beta):
            n = x.shape[0] * D
            colsum = jnp.dot(ones_row, x, preferred_element_type=jnp.float32)
            colsq = jnp.dot(ones_row, x * x,
                            preferred_element_type=jnp.float32)
            moments = jnp.concatenate([colsum, colsq], axis=0)
            pm = jnp.dot(moments, pool,
                         preferred_element_type=jnp.float32) / n
            mean = pm[0:1]
            var = jnp.maximum(pm[1:2] - mean * mean, 0.0)
            mi = jnp.concatenate([mean, lax.rsqrt(var + EPS)], axis=0)
            mi_e = jnp.dot(mi, poolt, preferred_element_type=jnp.float32)
            scale = mi_e[1:2] * gamma
            shift = beta - mi_e[0:1] * scale
            return x * scale + shift

        xn = batch_norm(x5, p0_ref[...], p0t_ref[...], g0_ref[...],
                        b0_ref[...])
        zcol = jnp.zeros((B, 1), jnp.float32)
        ls, ms, rs = [], [], []
        for i in range(CIN):
            xi = xn[:, i * D:(i + 1) * D]
            ms.append(xi)
            ls.append(jnp.concatenate([zcol, xi[:, :D - 1]], axis=1))
            rs.append(jnp.concatenate([xi[:, 1:], zcol], axis=1))
        outs = []
        for c in range(C):
            acc = None
            for i in range(CIN):
                t = (wc_scr[c, (3 * i) * 128 + 1] * ls[i]
                     + wc_scr[c, (3 * i + 1) * 128 + 1] * ms[i]
                     + wc_scr[c, (3 * i + 2) * 128 + 1] * rs[i])
                acc = t if acc is None else acc + t
            outs.append(acc)
        pre = jnp.concatenate(outs, axis=1)
        h1 = jnp.maximum(
            batch_norm(pre, p1_ref[...], p1t_ref[...], g1_ref[...],
                       b1_ref[...]), 0.0)
        h = jnp.dot(h1, w_fc_ref[...],
                    preferred_element_type=jnp.float32) + b_fc_ref[...]
        m2 = jnp.dot(ones_row, h, preferred_element_type=jnp.float32) / B
        q2 = jnp.dot(ones_row, h * h, preferred_element_type=jnp.float32) / B
        v2 = jnp.maximum(q2 - m2 * m2, 0.0)
        scale2 = lax.rsqrt(v2 + EPS) * g2_ref[...]
        shift2 = b2_ref[...] - m2 * scale2
        x_scr[...] = jnp.maximum(h * scale2 + shift2, 0.0)

    @pl.when(j > 0)
    def score_tile():
        t = jnp.tanh(emb_ref[...])
        s = lax.dot_general(
            x_scr[...], t,
            dimension_numbers=(((1,), (1,)), ((), ())),
            preferred_element_type=jnp.float32)
        o_ref[...] = s * pmask_ref[...]


def fused_forward(conv_band, triplets, t1, t2, his, w_fc1, b_fc1,
                  p0, p0t, g0, b0, p1, p1t, g1, b1, w_fc_t, b_fc, g2, b2,
                  embedding, emb_rel, partial_mask):
    B, D = t1.shape
    E = embedding.shape[0]
    te = min(ENT_TILE, E)
    nt = E // te
    assert nt * te == E

    smem = pl.BlockSpec(memory_space=pltpu.MemorySpace.SMEM)
    anyspace = pl.BlockSpec(memory_space=pltpu.MemorySpace.HBM)

    def const_spec(arr):
        nd = len(arr.shape)
        return pl.BlockSpec(arr.shape, lambda j, _n=nd: (0,) * _n)

    conv_args = (t1, t2, his, w_fc1, b_fc1,
                 p0, p0t, g0, b0, p1, p1t, g1, b1, w_fc_t, b_fc, g2, b2)
    in_specs = ([anyspace, smem]
                + [const_spec(a) for a in conv_args]
                + [anyspace, anyspace]
                + [pl.BlockSpec((te, D), lambda j: (jnp.maximum(j - 1, 0), 0)),
                   pl.BlockSpec((B, te), lambda j: (0, jnp.maximum(j - 1, 0)))])
    out_spec = pl.BlockSpec((B, te), lambda j: (0, jnp.maximum(j - 1, 0)))
    return pl.pallas_call(
        _fused_kernel,
        out_shape=jax.ShapeDtypeStruct((B, E), jnp.float32),
        grid=(1 + nt,),
        in_specs=in_specs,
        out_specs=out_spec,
        scratch_shapes=[pltpu.VMEM((B, D), jnp.float32),
                        pltpu.VMEM((B, D), jnp.float32),
                        pltpu.VMEM((B, D), jnp.float32),
                        pltpu.SMEM((16, KSIZE * CIN * 128), jnp.float32),
                        pltpu.SemaphoreType.DMA],
        compiler_params=pltpu.CompilerParams(
            dimension_semantics=("arbitrary",),
            vmem_limit_bytes=64 * 1024 * 1024),
        cost_estimate=pl.CostEstimate(
            flops=2 * B * D * E + 2 * B * (CIN * D) * 16 * 3,
            transcendentals=(E + B) * D,
            bytes_accessed=(E * D + 2 * B * E + B * D) * 4),
    )(conv_band, triplets, *conv_args, embedding, emb_rel,
      embedding, partial_mask)


def kernel(w_fc1, b_fc1, conv_band, b_conv_e, p0, p0t, p1, p1t,
           g0_e, b0_e, g1_e, b1_e, w_fc_t, b_fc, g2, b2,
           embedding, emb_rel, emb_time0, emb_time1,
           triplets, e_r_his_emb, partial):
    return fused_forward(conv_band, triplets, emb_time0, emb_time1,
                         e_r_his_emb, w_fc1, b_fc1, p0, p0t, g0_e, b0_e,
                         p1, p1t, g1_e, b1_e, w_fc_t, b_fc, g2, b2,
                         embedding, emb_rel, partial)


# speedup vs baseline: 1.0212x; 1.0212x over previous
import jax
import jax.numpy as jnp
from jax import lax
from jax.experimental import pallas as pl
from jax.experimental.pallas import tpu as pltpu

CIN = 5
KSIZE = 3
EPS = 1e-5
ENT_TILE = 4096


def _fused_kernel(band_ref, trip_ref,
                  t1_ref, t2_ref, his_ref,
                  w_fc1_ref, b_fc1_ref,
                  p0_ref, p0t_ref, g0_ref, b0_ref,
                  p1_ref, p1t_ref, g1_ref, b1_ref,
                  w_fc_ref, b_fc_ref, g2_ref, b2_ref,
                  emb_any, rel_any,
                  emb_ref, pmask_ref,
                  o_ref,
                  e1_scr, rel_scr, x_scr, wc_scr, wfc_scr, dma_sem,
                  wfc_sem):
    j = pl.program_id(0)
    B, D = t1_ref.shape
    C = p1_ref.shape[1]

    @pl.when(j == 0)
    def conv_path():
        pltpu.make_async_copy(w_fc_ref, wfc_scr, wfc_sem).start()
        for c in range(C):
            for i in range(CIN):
                for k in range(KSIZE):
                    pltpu.make_async_copy(
                        band_ref.at[pl.ds(i * D + k, 1), pl.ds(c * D, 128)],
                        wc_scr.at[pl.ds(c, 1), pl.ds((3 * i + k) * 128, 128)],
                        dma_sem).start()
        def issue(i, carry):
            pltpu.make_async_copy(
                emb_any.at[pl.ds(trip_ref[i, 0], 1)],
                e1_scr.at[pl.ds(i, 1)], dma_sem).start()
            pltpu.make_async_copy(
                rel_any.at[pl.ds(trip_ref[i, 1], 1)],
                rel_scr.at[pl.ds(i, 1)], dma_sem).start()
            return carry
        lax.fori_loop(0, B, issue, 0)

        ones_row = jnp.ones((1, B), jnp.float32)
        his = jnp.dot(his_ref[...], w_fc1_ref[...],
                      preferred_element_type=jnp.float32) + b_fc1_ref[...]

        pltpu.make_async_copy(emb_any.at[pl.ds(0, B)], e1_scr, dma_sem).wait()
        pltpu.make_async_copy(rel_any.at[pl.ds(0, B)], rel_scr, dma_sem).wait()
        pltpu.make_async_copy(
            band_ref.at[pl.ds(0, 16), pl.ds(0, KSIZE * CIN * 128)], wc_scr,
            dma_sem).wait()
        x5 = jnp.concatenate(
            [jnp.tanh(e1_scr[...]), rel_scr[...], t1_ref[...], t2_ref[...],
             his], axis=1)

        def batch_norm(x, pool, poolt, gamma, beta):
            n = x.shape[0] * D
            colsum = jnp.dot(ones_row, x, preferred_element_type=jnp.float32)
            colsq = jnp.dot(ones_row, x * x,
                            preferred_element_type=jnp.float32)
            moments = jnp.concatenate([colsum, colsq], axis=0)
            pm = jnp.dot(moments, pool,
                         preferred_element_type=jnp.float32) / n
            mean = pm[0:1]
            var = jnp.maximum(pm[1:2] - mean * mean, 0.0)
            mi = jnp.concatenate([mean, lax.rsqrt(var + EPS)], axis=0)
            mi_e = jnp.dot(mi, poolt, preferred_element_type=jnp.float32)
            scale = mi_e[1:2] * gamma
            shift = beta - mi_e[0:1] * scale
            return x * scale + shift

        xn = batch_norm(x5, p0_ref[...], p0t_ref[...], g0_ref[...],
                        b0_ref[...])
        zcol = jnp.zeros((B, 1), jnp.float32)
        ls, ms, rs = [], [], []
        for i in range(CIN):
            xi = xn[:, i * D:(i + 1) * D]
            ms.append(xi)
            ls.append(jnp.concatenate([zcol, xi[:, :D - 1]], axis=1))
            rs.append(jnp.concatenate([xi[:, 1:], zcol], axis=1))
        outs = []
        for c in range(C):
            acc = None
            for i in range(CIN):
                t = (wc_scr[c, (3 * i) * 128 + 1] * ls[i]
                     + wc_scr[c, (3 * i + 1) * 128 + 1] * ms[i]
                     + wc_scr[c, (3 * i + 2) * 128 + 1] * rs[i])
                acc = t if acc is None else acc + t
            outs.append(acc)
        pre = jnp.concatenate(outs, axis=1)
        h1 = jnp.maximum(
            batch_norm(pre, p1_ref[...], p1t_ref[...], g1_ref[...],
                       b1_ref[...]), 0.0)
        pltpu.make_async_copy(w_fc_ref, wfc_scr, wfc_sem).wait()
        h = jnp.dot(h1, wfc_scr[...],
                    preferred_element_type=jnp.float32) + b_fc_ref[...]
        m2 = jnp.dot(ones_row, h, preferred_element_type=jnp.float32) / B
        q2 = jnp.dot(ones_row, h * h, preferred_element_type=jnp.float32) / B
        v2 = jnp.maximum(q2 - m2 * m2, 0.0)
        scale2 = lax.rsqrt(v2 + EPS) * g2_ref[...]
        shift2 = b2_ref[...] - m2 * scale2
        x_scr[...] = jnp.maximum(h * scale2 + shift2, 0.0)

    @pl.when(j > 0)
    def score_tile():
        t = jnp.tanh(emb_ref[...])
        s = lax.dot_general(
            x_scr[...], t,
            dimension_numbers=(((1,), (1,)), ((), ())),
            preferred_element_type=jnp.float32)
        o_ref[...] = s * pmask_ref[...]


def fused_forward(conv_band, triplets, t1, t2, his, w_fc1, b_fc1,
                  p0, p0t, g0, b0, p1, p1t, g1, b1, w_fc_t, b_fc, g2, b2,
                  embedding, emb_rel, partial_mask):
    B, D = t1.shape
    E = embedding.shape[0]
    te = min(ENT_TILE, E)
    nt = E // te
    assert nt * te == E

    smem = pl.BlockSpec(memory_space=pltpu.MemorySpace.SMEM)
    anyspace = pl.BlockSpec(memory_space=pltpu.MemorySpace.HBM)

    def const_spec(arr):
        nd = len(arr.shape)
        return pl.BlockSpec(arr.shape, lambda j, _n=nd: (0,) * _n)

    conv_args = (t1, t2, his, w_fc1, b_fc1,
                 p0, p0t, g0, b0, p1, p1t, g1, b1, w_fc_t, b_fc, g2, b2)
    in_specs = ([anyspace, smem]
                + [anyspace if a is w_fc_t else const_spec(a)
                   for a in conv_args]
                + [anyspace, anyspace]
                + [pl.BlockSpec((te, D), lambda j: (jnp.maximum(j - 1, 0), 0)),
                   pl.BlockSpec((B, te), lambda j: (0, jnp.maximum(j - 1, 0)))])
    out_spec = pl.BlockSpec((B, te), lambda j: (0, jnp.maximum(j - 1, 0)))
    return pl.pallas_call(
        _fused_kernel,
        out_shape=jax.ShapeDtypeStruct((B, E), jnp.float32),
        grid=(1 + nt,),
        in_specs=in_specs,
        out_specs=out_spec,
        scratch_shapes=[pltpu.VMEM((B, D), jnp.float32),
                        pltpu.VMEM((B, D), jnp.float32),
                        pltpu.VMEM((B, D), jnp.float32),
                        pltpu.SMEM((16, KSIZE * CIN * 128), jnp.float32),
                        pltpu.VMEM(w_fc_t.shape, jnp.float32),
                        pltpu.SemaphoreType.DMA,
                        pltpu.SemaphoreType.DMA],
        compiler_params=pltpu.CompilerParams(
            dimension_semantics=("arbitrary",),
            vmem_limit_bytes=64 * 1024 * 1024),
        cost_estimate=pl.CostEstimate(
            flops=2 * B * D * E + 2 * B * (CIN * D) * 16 * 3,
            transcendentals=(E + B) * D,
            bytes_accessed=(E * D + 2 * B * E + B * D) * 4),
    )(conv_band, triplets, *conv_args, embedding, emb_rel,
      embedding, partial_mask)


def kernel(w_fc1, b_fc1, conv_band, b_conv_e, p0, p0t, p1, p1t,
           g0_e, b0_e, g1_e, b1_e, w_fc_t, b_fc, g2, b2,
           embedding, emb_rel, emb_time0, emb_time1,
           triplets, e_r_his_emb, partial):
    return fused_forward(conv_band, triplets, emb_time0, emb_time1,
                         e_r_his_emb, w_fc1, b_fc1, p0, p0t, g0_e, b0_e,
                         p1, p1t, g1_e, b1_e, w_fc_t, b_fc, g2, b2,
                         embedding, emb_rel, partial)


# speedup vs baseline: 1.0320x; 1.0106x over previous
import jax
import jax.numpy as jnp
from jax import lax
from jax.experimental import pallas as pl
from jax.experimental.pallas import tpu as pltpu

CIN = 5
KSIZE = 3
EPS = 1e-5
ENT_TILE = 4096


def _fused_kernel(band_ref, trip_ref,
                  t1_ref, t2_ref, his_ref,
                  w_fc1_ref, b_fc1_ref,
                  p0_ref, p0t_ref, g0_ref, b0_ref,
                  p1_ref, p1t_ref, g1_ref, b1_ref,
                  w_fc_ref, b_fc_ref, g2_ref, b2_ref,
                  emb_any, rel_any,
                  emb_ref, pmask_ref,
                  o_ref,
                  e1_scr, rel_scr, x_scr, wc_scr, wfc_scr, dma_sem,
                  wfc_sem):
    j = pl.program_id(0)
    B, D = t1_ref.shape
    C = p1_ref.shape[1]

    @pl.when(j == 0)
    def conv_path():
        pltpu.make_async_copy(w_fc_ref, wfc_scr, wfc_sem).start()
        for c in range(C):
            for i in range(CIN):
                for k in range(KSIZE):
                    pltpu.make_async_copy(
                        band_ref.at[pl.ds(i * D + k, 1), pl.ds(c * D, 128)],
                        wc_scr.at[pl.ds(c, 1), pl.ds((3 * i + k) * 128, 128)],
                        dma_sem).start()
        def issue(i, carry):
            pltpu.make_async_copy(
                emb_any.at[pl.ds(trip_ref[i, 0], 1)],
                e1_scr.at[pl.ds(i, 1)], dma_sem).start()
            pltpu.make_async_copy(
                rel_any.at[pl.ds(trip_ref[i, 1], 1)],
                rel_scr.at[pl.ds(i, 1)], dma_sem).start()
            return carry
        lax.fori_loop(0, B, issue, 0)

        ones_row = jnp.ones((1, B), jnp.float32)
        his = jnp.dot(his_ref[...], w_fc1_ref[...],
                      preferred_element_type=jnp.float32) + b_fc1_ref[...]

        pltpu.make_async_copy(emb_any.at[pl.ds(0, B)], e1_scr, dma_sem).wait()
        pltpu.make_async_copy(rel_any.at[pl.ds(0, B)], rel_scr, dma_sem).wait()
        pltpu.make_async_copy(
            band_ref.at[pl.ds(0, 16), pl.ds(0, KSIZE * CIN * 128)], wc_scr,
            dma_sem).wait()
        x5 = jnp.concatenate(
            [jnp.tanh(e1_scr[...]), rel_scr[...], t1_ref[...], t2_ref[...],
             his], axis=1)

        def batch_norm(x, pool, poolt, gamma, beta):
            n = x.shape[0] * D
            colsum = jnp.dot(ones_row, x, preferred_element_type=jnp.float32)
            colsq = jnp.dot(ones_row, x * x,
                            preferred_element_type=jnp.float32)
            moments = jnp.concatenate([colsum, colsq], axis=0)
            pm = jnp.dot(moments, pool,
                         preferred_element_type=jnp.float32) / n
            mean = pm[0:1]
            var = jnp.maximum(pm[1:2] - mean * mean, 0.0)
            mi = jnp.concatenate([mean, lax.rsqrt(var + EPS)], axis=0)
            mi_e = jnp.dot(mi, poolt, preferred_element_type=jnp.float32)
            scale = mi_e[1:2] * gamma
            shift = beta - mi_e[0:1] * scale
            return x * scale + shift

        xn = batch_norm(x5, p0_ref[...], p0t_ref[...], g0_ref[...],
                        b0_ref[...])
        zcol = jnp.zeros((B, 1), jnp.float32)
        ls, ms, rs = [], [], []
        for i in range(CIN):
            xi = xn[:, i * D:(i + 1) * D]
            ms.append(xi)
            ls.append(jnp.concatenate([zcol, xi[:, :D - 1]], axis=1))
            rs.append(jnp.concatenate([xi[:, 1:], zcol], axis=1))
        outs = []
        for c in range(C):
            acc = None
            for i in range(CIN):
                t = (wc_scr[c, (3 * i) * 128 + 1] * ls[i]
                     + wc_scr[c, (3 * i + 1) * 128 + 1] * ms[i]
                     + wc_scr[c, (3 * i + 2) * 128 + 1] * rs[i])
                acc = t if acc is None else acc + t
            outs.append(acc)
        pre = jnp.concatenate(outs, axis=1)
        h1 = jnp.maximum(
            batch_norm(pre, p1_ref[...], p1t_ref[...], g1_ref[...],
                       b1_ref[...]), 0.0)
        pltpu.make_async_copy(w_fc_ref, wfc_scr, wfc_sem).wait()
        h = jnp.dot(h1, wfc_scr[...],
                    preferred_element_type=jnp.float32) + b_fc_ref[...]
        m2 = jnp.dot(ones_row, h, preferred_element_type=jnp.float32) / B
        q2 = jnp.dot(ones_row, h * h, preferred_element_type=jnp.float32) / B
        v2 = jnp.maximum(q2 - m2 * m2, 0.0)
        scale2 = lax.rsqrt(v2 + EPS) * g2_ref[...]
        shift2 = b2_ref[...] - m2 * scale2
        x_scr[...] = jnp.maximum(h * scale2 + shift2, 0.0)

    t = jnp.tanh(emb_ref[...])
    s = lax.dot_general(
        x_scr[...], t,
        dimension_numbers=(((1,), (1,)), ((), ())),
        preferred_element_type=jnp.float32)
    o_ref[...] = s * pmask_ref[...]


def fused_forward(conv_band, triplets, t1, t2, his, w_fc1, b_fc1,
                  p0, p0t, g0, b0, p1, p1t, g1, b1, w_fc_t, b_fc, g2, b2,
                  embedding, emb_rel, partial_mask):
    B, D = t1.shape
    E = embedding.shape[0]
    te = min(ENT_TILE, E)
    nt = E // te
    assert nt * te == E

    smem = pl.BlockSpec(memory_space=pltpu.MemorySpace.SMEM)
    anyspace = pl.BlockSpec(memory_space=pltpu.MemorySpace.HBM)

    def const_spec(arr):
        nd = len(arr.shape)
        return pl.BlockSpec(arr.shape, lambda j, _n=nd: (0,) * _n)

    conv_args = (t1, t2, his, w_fc1, b_fc1,
                 p0, p0t, g0, b0, p1, p1t, g1, b1, w_fc_t, b_fc, g2, b2)
    in_specs = ([anyspace, smem]
                + [anyspace if a is w_fc_t else const_spec(a)
                   for a in conv_args]
                + [anyspace, anyspace]
                + [pl.BlockSpec((te, D), lambda j: (j, 0)),
                   pl.BlockSpec((B, te), lambda j: (0, j))])
    out_spec = pl.BlockSpec((B, te), lambda j: (0, j))
    return pl.pallas_call(
        _fused_kernel,
        out_shape=jax.ShapeDtypeStruct((B, E), jnp.float32),
        grid=(nt,),
        in_specs=in_specs,
        out_specs=out_spec,
        scratch_shapes=[pltpu.VMEM((B, D), jnp.float32),
                        pltpu.VMEM((B, D), jnp.float32),
                        pltpu.VMEM((B, D), jnp.float32),
                        pltpu.SMEM((16, KSIZE * CIN * 128), jnp.float32),
                        pltpu.VMEM(w_fc_t.shape, jnp.float32),
                        pltpu.SemaphoreType.DMA,
                        pltpu.SemaphoreType.DMA],
        compiler_params=pltpu.CompilerParams(
            dimension_semantics=("arbitrary",),
            vmem_limit_bytes=64 * 1024 * 1024),
        cost_estimate=pl.CostEstimate(
            flops=2 * B * D * E + 2 * B * (CIN * D) * 16 * 3,
            transcendentals=(E + B) * D,
            bytes_accessed=(E * D + 2 * B * E + B * D) * 4),
    )(conv_band, triplets, *conv_args, embedding, emb_rel,
      embedding, partial_mask)


def kernel(w_fc1, b_fc1, conv_band, b_conv_e, p0, p0t, p1, p1t,
           g0_e, b0_e, g1_e, b1_e, w_fc_t, b_fc, g2, b2,
           embedding, emb_rel, emb_time0, emb_time1,
           triplets, e_r_his_emb, partial):
    return fused_forward(conv_band, triplets, emb_time0, emb_time1,
                         e_r_his_emb, w_fc1, b_fc1, p0, p0t, g0_e, b0_e,
                         p1, p1t, g1_e, b1_e, w_fc_t, b_fc, g2, b2,
                         embedding, emb_rel, partial)
